# baseline (speedup 1.0000x reference)
import functools

import jax
import jax.numpy as jnp
from jax import lax
from jax.experimental import pallas as pl
from jax.experimental.pallas import tpu as pltpu
from jax.experimental.pallas import tpu_sc as plsc

_NCHAN, _NPIX, _NNZ = 8, 1024, 2_000_000
_FLAT = _NCHAN * _NPIX * _NPIX
_NC, _NS = 2, 16
_NW = _NC * _NS
_B = 4096
_NBLK = 16
_WORK = _B * _NBLK
_W = 24576
_L = 16
_UNROLL = 8


def _make_gather():
    mesh = plsc.VectorSubcoreMesh(core_axis_name="c", subcore_axis_name="s")

    @functools.partial(
        pl.kernel,
        mesh=mesh,
        out_type=(
            jax.ShapeDtypeStruct((_NNZ,), jnp.float32),
            jax.ShapeDtypeStruct((_NNZ,), jnp.float32),
        ),
        scratch_types=[
            pltpu.VMEM((_B,), jnp.int32),
            pltpu.VMEM((_B,), jnp.int32),
            pltpu.VMEM((_B,), jnp.float32),
            pltpu.VMEM((_B,), jnp.float32),
            pltpu.VMEM_SHARED((_NS * _W,), jnp.float32),
            pltpu.VMEM_SHARED((_NS * _W,), jnp.float32),
            pltpu.VMEM_SHARED((_NS * _W,), jnp.float32),
            pltpu.VMEM_SHARED((_NS * _W,), jnp.float32),
            pltpu.SemaphoreType.DMA,
            pltpu.SemaphoreType.DMA,
            pltpu.SemaphoreType.DMA,
            pltpu.SemaphoreType.DMA,
            pltpu.SemaphoreType.DMA,
            pltpu.SemaphoreType.DMA,
            pltpu.SemaphoreType.DMA,
            pltpu.SemaphoreType.DMA,
        ],
    )
    def gather_kernel(re_hbm, im_hbm, idx_hbm, out_re, out_im,
                      idx0, idx1, ore_v, oim_v,
                      wre0, wim0, wre1, wim1,
                      sidx0, sidx1, swre0, swim0, swre1, swim1,
                      sg_re, sg_im):
        idx_b = (idx0, idx1)
        wre_b = (wre0, wre1)
        wim_b = (wim0, wim1)
        sidx = (sidx0, sidx1)
        swre = (swre0, swre1)
        swim = (swim0, swim1)

        cid = lax.axis_index("c")
        sid = lax.axis_index("s")
        wid = sid * _NC + cid
        base = jnp.minimum(wid * _WORK, _NNZ - _WORK)
        wbase = sid * _W
        wbase_dma = pl.multiple_of(wbase, 128)

        def stage(g, p, active):
            iv = idx_b[p]
            lo = iv[pl.ds(0, _L)][0]
            hi = iv[pl.ds(_B - _L, _L)][_L - 1]
            wstart = jnp.minimum((lo >> 7) << 7, _FLAT - _W)
            fast = (hi - wstart) < _W

            @pl.when(fast & active)
            def _():
                pltpu.async_copy(
                    re_hbm.at[pl.ds(pl.multiple_of(wstart, 128), _W)],
                    wre_b[p].at[pl.ds(wbase_dma, _W)], swre[p])
                pltpu.async_copy(
                    im_hbm.at[pl.ds(pl.multiple_of(wstart, 128), _W)],
                    wim_b[p].at[pl.ds(wbase_dma, _W)], swim[p])

                shift = wstart - wbase

                def inner(o, c):
                    for u in range(_UNROLL):
                        i = (o * _UNROLL + u) * _L
                        iv[pl.ds(i, _L)] = iv[pl.ds(i, _L)] - shift
                    return c

                lax.fori_loop(0, _B // (_L * _UNROLL), inner, 0)

            return fast

        def drain(g, p, fast):
            @pl.when(fast)
            def _():
                pltpu.make_async_copy(
                    re_hbm.at[pl.ds(0, _W)],
                    wre_b[p].at[pl.ds(wbase_dma, _W)], swre[p]).wait()
                pltpu.make_async_copy(
                    im_hbm.at[pl.ds(0, _W)],
                    wim_b[p].at[pl.ds(wbase_dma, _W)], swim[p]).wait()
                cg_re = pltpu.async_copy(wre_b[p].at[idx_b[p]], ore_v, sg_re)
                cg_im = pltpu.async_copy(wim_b[p].at[idx_b[p]], oim_v, sg_im)
                cg_re.wait()
                cg_im.wait()

            @pl.when(jnp.logical_not(fast))
            def _():
                cg_re = pltpu.async_copy(re_hbm.at[idx_b[p]], ore_v, sg_re)
                cg_im = pltpu.async_copy(im_hbm.at[idx_b[p]], oim_v, sg_im)
                cg_re.wait()
                cg_im.wait()

            off = base + g * _B
            pltpu.sync_copy(ore_v, out_re.at[pl.ds(off, _B)])
            pltpu.sync_copy(oim_v, out_im.at[pl.ds(off, _B)])

        pltpu.sync_copy(idx_hbm.at[pl.ds(base, _B)], idx0)
        fast0 = stage(0, 0, jnp.bool_(True))
        pltpu.async_copy(idx_hbm.at[pl.ds(base + _B, _B)], idx1, sidx1)

        def body(g, fast_g):
            for par in range(2):
                gg = g * 2 + par
                p = par
                q = 1 - par
                have_next = gg + 1 < _NBLK

                @pl.when(have_next)
                def _():
                    pltpu.make_async_copy(
                        idx_hbm.at[pl.ds(0, _B)], idx_b[q], sidx[q]).wait()

                fast_next = stage(gg + 1, q, have_next)

                drain(gg, p, fast_g)

                @pl.when(gg + 2 < _NBLK)
                def _():
                    pltpu.async_copy(
                        idx_hbm.at[pl.ds(base + (gg + 2) * _B, _B)],
                        idx_b[p], sidx[p])

                fast_g = fast_next
            return fast_g

        lax.fori_loop(0, _NBLK // 2, body, fast0)

    return gather_kernel


def kernel(modelVisibilityCube_real, modelVisibilityCube_imag, mask_idx):
    re_flat = modelVisibilityCube_real.reshape(-1)
    im_flat = modelVisibilityCube_imag.reshape(-1)
    re, im = _make_gather()(re_flat, im_flat, mask_idx)
    return re.astype(jnp.complex64) + 1j * im.astype(jnp.complex64)


# speedup vs baseline: 1.0409x; 1.0409x over previous
import functools

import jax
import jax.numpy as jnp
from jax import lax
from jax.experimental import pallas as pl
from jax.experimental.pallas import tpu as pltpu
from jax.experimental.pallas import tpu_sc as plsc

_NCHAN, _NPIX, _NNZ = 8, 1024, 2_000_000
_FLAT = _NCHAN * _NPIX * _NPIX
_NC, _NS = 2, 16
_NW = _NC * _NS
_B = 4096
_NBLK = 16
_WORK = _B * _NBLK
_W = 19456
_L = 16
_UNROLL = 8


def _make_gather():
    mesh = plsc.VectorSubcoreMesh(core_axis_name="c", subcore_axis_name="s")

    @functools.partial(
        pl.kernel,
        mesh=mesh,
        out_type=(
            jax.ShapeDtypeStruct((_NNZ,), jnp.float32),
            jax.ShapeDtypeStruct((_NNZ,), jnp.float32),
        ),
        scratch_types=[
            pltpu.VMEM((_B,), jnp.int32),
            pltpu.VMEM((_B,), jnp.int32),
            pltpu.VMEM((_B,), jnp.float32),
            pltpu.VMEM((_B,), jnp.float32),
            pltpu.VMEM_SHARED((_NS * _W,), jnp.float32),
            pltpu.VMEM_SHARED((_NS * _W,), jnp.float32),
            pltpu.VMEM_SHARED((_NS * _W,), jnp.float32),
            pltpu.VMEM_SHARED((_NS * _W,), jnp.float32),
            pltpu.SemaphoreType.DMA,
            pltpu.SemaphoreType.DMA,
            pltpu.SemaphoreType.DMA,
            pltpu.SemaphoreType.DMA,
            pltpu.SemaphoreType.DMA,
            pltpu.SemaphoreType.DMA,
            pltpu.SemaphoreType.DMA,
            pltpu.SemaphoreType.DMA,
        ],
    )
    def gather_kernel(re_hbm, im_hbm, idx_hbm, out_re, out_im,
                      idx0, idx1, ore_v, oim_v,
                      wre0, wim0, wre1, wim1,
                      sidx0, sidx1, swre0, swim0, swre1, swim1,
                      sg_re, sg_im):
        idx_b = (idx0, idx1)
        wre_b = (wre0, wre1)
        wim_b = (wim0, wim1)
        sidx = (sidx0, sidx1)
        swre = (swre0, swre1)
        swim = (swim0, swim1)

        cid = lax.axis_index("c")
        sid = lax.axis_index("s")
        wid = sid * _NC + cid
        base = jnp.minimum(wid * _WORK, _NNZ - _WORK)
        wbase = sid * _W
        wbase_dma = pl.multiple_of(wbase, 128)

        def stage(g, p, active):
            iv = idx_b[p]
            lo = iv[pl.ds(0, _L)][0]
            hi = iv[pl.ds(_B - _L, _L)][_L - 1]
            wstart = jnp.minimum((lo >> 7) << 7, _FLAT - _W)
            fast = (hi - wstart) < _W

            @pl.when(fast & active)
            def _():
                pltpu.async_copy(
                    re_hbm.at[pl.ds(pl.multiple_of(wstart, 128), _W)],
                    wre_b[p].at[pl.ds(wbase_dma, _W)], swre[p])
                pltpu.async_copy(
                    im_hbm.at[pl.ds(pl.multiple_of(wstart, 128), _W)],
                    wim_b[p].at[pl.ds(wbase_dma, _W)], swim[p])

                shift = wstart - wbase

                def inner(o, c):
                    for u in range(_UNROLL):
                        i = (o * _UNROLL + u) * _L
                        iv[pl.ds(i, _L)] = iv[pl.ds(i, _L)] - shift
                    return c

                lax.fori_loop(0, _B // (_L * _UNROLL), inner, 0)

            return fast

        def drain(g, p, fast):
            @pl.when(fast)
            def _():
                pltpu.make_async_copy(
                    re_hbm.at[pl.ds(0, _W)],
                    wre_b[p].at[pl.ds(wbase_dma, _W)], swre[p]).wait()
                pltpu.make_async_copy(
                    im_hbm.at[pl.ds(0, _W)],
                    wim_b[p].at[pl.ds(wbase_dma, _W)], swim[p]).wait()
                cg_re = pltpu.async_copy(wre_b[p].at[idx_b[p]], ore_v, sg_re)
                cg_im = pltpu.async_copy(wim_b[p].at[idx_b[p]], oim_v, sg_im)
                cg_re.wait()
                cg_im.wait()

            @pl.when(jnp.logical_not(fast))
            def _():
                cg_re = pltpu.async_copy(re_hbm.at[idx_b[p]], ore_v, sg_re)
                cg_im = pltpu.async_copy(im_hbm.at[idx_b[p]], oim_v, sg_im)
                cg_re.wait()
                cg_im.wait()

            off = base + g * _B
            pltpu.sync_copy(ore_v, out_re.at[pl.ds(off, _B)])
            pltpu.sync_copy(oim_v, out_im.at[pl.ds(off, _B)])

        pltpu.sync_copy(idx_hbm.at[pl.ds(base, _B)], idx0)
        fast0 = stage(0, 0, jnp.bool_(True))
        pltpu.async_copy(idx_hbm.at[pl.ds(base + _B, _B)], idx1, sidx1)

        def body(g, fast_g):
            for par in range(2):
                gg = g * 2 + par
                p = par
                q = 1 - par
                have_next = gg + 1 < _NBLK

                @pl.when(have_next)
                def _():
                    pltpu.make_async_copy(
                        idx_hbm.at[pl.ds(0, _B)], idx_b[q], sidx[q]).wait()

                fast_next = stage(gg + 1, q, have_next)

                drain(gg, p, fast_g)

                @pl.when(gg + 2 < _NBLK)
                def _():
                    pltpu.async_copy(
                        idx_hbm.at[pl.ds(base + (gg + 2) * _B, _B)],
                        idx_b[p], sidx[p])

                fast_g = fast_next
            return fast_g

        lax.fori_loop(0, _NBLK // 2, body, fast0)

    return gather_kernel


def kernel(modelVisibilityCube_real, modelVisibilityCube_imag, mask_idx):
    re_flat = modelVisibilityCube_real.reshape(-1)
    im_flat = modelVisibilityCube_imag.reshape(-1)
    re, im = _make_gather()(re_flat, im_flat, mask_idx)
    return jax.lax.complex(re, im)


# speedup vs baseline: 1.0520x; 1.0107x over previous
import functools

import jax
import jax.numpy as jnp
from jax import lax
from jax.experimental import pallas as pl
from jax.experimental.pallas import tpu as pltpu
from jax.experimental.pallas import tpu_sc as plsc

_NCHAN, _NPIX, _NNZ = 8, 1024, 2_000_000
_FLAT = _NCHAN * _NPIX * _NPIX
_NC, _NS = 2, 16
_NW = _NC * _NS
_B = 4096
_NBLK = 16
_WORK = _B * _NBLK
_W = 19456
_L = 16
_UNROLL = 8


def _make_gather():
    mesh = plsc.VectorSubcoreMesh(core_axis_name="c", subcore_axis_name="s")

    @functools.partial(
        pl.kernel,
        mesh=mesh,
        out_type=(
            jax.ShapeDtypeStruct((_NNZ,), jnp.float32),
            jax.ShapeDtypeStruct((_NNZ,), jnp.float32),
        ),
        scratch_types=[
            pltpu.VMEM((_B,), jnp.int32),
            pltpu.VMEM((_B,), jnp.int32),
            pltpu.VMEM((2 * _B,), jnp.int32),
            pltpu.VMEM((2 * _B,), jnp.int32),
            pltpu.VMEM((2 * _B,), jnp.float32),
            pltpu.VMEM((2 * _B,), jnp.float32),
            pltpu.VMEM_SHARED((_NS * 2 * _W,), jnp.float32),
            pltpu.VMEM_SHARED((_NS * 2 * _W,), jnp.float32),
            pltpu.SemaphoreType.DMA,
            pltpu.SemaphoreType.DMA,
            pltpu.SemaphoreType.DMA,
            pltpu.SemaphoreType.DMA,
            pltpu.SemaphoreType.DMA,
            pltpu.SemaphoreType.DMA,
            pltpu.SemaphoreType.DMA,
            pltpu.SemaphoreType.DMA,
            pltpu.SemaphoreType.DMA,
        ],
    )
    def gather_kernel(re_hbm, im_hbm, idx_hbm, out_re, out_im,
                      idx0, idx1, loc0, loc1, pair0, pair1,
                      win0, win1,
                      sidx0, sidx1, swin0, swin1, sg,
                      sst_re0, sst_re1, sst_im0, sst_im1):
        idx_b = (idx0, idx1)
        loc_b = (loc0, loc1)
        pair_b = (pair0, pair1)
        win_b = (win0, win1)
        sidx = (sidx0, sidx1)
        swin = (swin0, swin1)
        sst_re = (sst_re0, sst_re1)
        sst_im = (sst_im0, sst_im1)

        cid = lax.axis_index("c")
        sid = lax.axis_index("s")
        wid = sid * _NC + cid
        base = jnp.minimum(wid * _WORK, _NNZ - _WORK)
        wbase = sid * 2 * _W
        wbase_dma = pl.multiple_of(wbase, 128)
        wbase_im_dma = pl.multiple_of(wbase + _W, 128)

        def stage(g, p, active):
            iv = idx_b[p]
            lv = loc_b[p]
            lo = iv[pl.ds(0, _L)][0]
            hi = iv[pl.ds(_B - _L, _L)][_L - 1]
            wstart = jnp.minimum((lo >> 7) << 7, _FLAT - _W)
            fast = (hi - wstart) < _W

            @pl.when(fast & active)
            def _():
                pltpu.async_copy(
                    re_hbm.at[pl.ds(pl.multiple_of(wstart, 128), _W)],
                    win_b[p].at[pl.ds(wbase_dma, _W)], swin[p])
                pltpu.async_copy(
                    im_hbm.at[pl.ds(pl.multiple_of(wstart, 128), _W)],
                    win_b[p].at[pl.ds(wbase_im_dma, _W)], swin[p])

                shift = wstart - wbase

                def inner(o, c):
                    for u in range(_UNROLL):
                        i = (o * _UNROLL + u) * _L
                        a = iv[pl.ds(i, _L)] - shift
                        lv[pl.ds(i, _L)] = a
                        lv[pl.ds(_B + i, _L)] = a + _W
                    return c

                lax.fori_loop(0, _B // (_L * _UNROLL), inner, 0)

            return fast

        def drain(g, p, fast):
            @pl.when(g >= 2)
            def _():
                pltpu.make_async_copy(
                    pair_b[p].at[pl.ds(0, _B)],
                    out_re.at[pl.ds(0, _B)], sst_re[p]).wait()
                pltpu.make_async_copy(
                    pair_b[p].at[pl.ds(_B, _B)],
                    out_im.at[pl.ds(0, _B)], sst_im[p]).wait()

            @pl.when(fast)
            def _():
                pltpu.make_async_copy(
                    re_hbm.at[pl.ds(0, _W)],
                    win_b[p].at[pl.ds(wbase_dma, _W)], swin[p]).wait()
                pltpu.make_async_copy(
                    im_hbm.at[pl.ds(0, _W)],
                    win_b[p].at[pl.ds(wbase_im_dma, _W)], swin[p]).wait()
                pltpu.async_copy(win_b[p].at[loc_b[p]], pair_b[p], sg).wait()

            @pl.when(jnp.logical_not(fast))
            def _():
                cg_re = pltpu.async_copy(
                    re_hbm.at[idx_b[p]], pair_b[p].at[pl.ds(0, _B)], sg)
                cg_im = pltpu.async_copy(
                    im_hbm.at[idx_b[p]], pair_b[p].at[pl.ds(_B, _B)], sg)
                cg_re.wait()
                cg_im.wait()

            off = base + g * _B
            pltpu.async_copy(
                pair_b[p].at[pl.ds(0, _B)], out_re.at[pl.ds(off, _B)],
                sst_re[p])
            pltpu.async_copy(
                pair_b[p].at[pl.ds(_B, _B)], out_im.at[pl.ds(off, _B)],
                sst_im[p])

        pltpu.sync_copy(idx_hbm.at[pl.ds(base, _B)], idx0)
        fast0 = stage(0, 0, jnp.bool_(True))
        pltpu.async_copy(idx_hbm.at[pl.ds(base + _B, _B)], idx1, sidx1)

        def body(g, fast_g):
            for par in range(2):
                gg = g * 2 + par
                p = par
                q = 1 - par
                have_next = gg + 1 < _NBLK

                @pl.when(have_next)
                def _():
                    pltpu.make_async_copy(
                        idx_hbm.at[pl.ds(0, _B)], idx_b[q], sidx[q]).wait()

                fast_next = stage(gg + 1, q, have_next)

                drain(gg, p, fast_g)

                @pl.when(gg + 2 < _NBLK)
                def _():
                    pltpu.async_copy(
                        idx_hbm.at[pl.ds(base + (gg + 2) * _B, _B)],
                        idx_b[p], sidx[p])

                fast_g = fast_next
            return fast_g

        lax.fori_loop(0, _NBLK // 2, body, fast0)

        for p in range(2):
            pltpu.make_async_copy(
                pair_b[p].at[pl.ds(0, _B)],
                out_re.at[pl.ds(0, _B)], sst_re[p]).wait()
            pltpu.make_async_copy(
                pair_b[p].at[pl.ds(_B, _B)],
                out_im.at[pl.ds(0, _B)], sst_im[p]).wait()

    return gather_kernel


def kernel(modelVisibilityCube_real, modelVisibilityCube_imag, mask_idx):
    re_flat = modelVisibilityCube_real.reshape(-1)
    im_flat = modelVisibilityCube_imag.reshape(-1)
    re, im = _make_gather()(re_flat, im_flat, mask_idx)
    return jax.lax.complex(re, im)
